# baseline (speedup 1.0000x reference)
import functools

import jax
import jax.numpy as jnp
from jax import lax
from jax.experimental import pallas as pl
from jax.experimental.pallas import tpu as pltpu
from jax.experimental.pallas import tpu_sc as plsc

VOCAB = 100000
EMBED = 64
B = 4096
L = 200

_info = plsc.get_sparse_core_info()
NC, NS = _info.num_cores, _info.num_subcores
NW = NC * NS
ROWS_W = B // NW
CB = 4
NCHUNK = ROWS_W // CB
NBUF = 2
NG = NCHUNK // NBUF

_mesh = plsc.VectorSubcoreMesh(core_axis_name="c", subcore_axis_name="s")


@functools.partial(
    pl.kernel,
    mesh=_mesh,
    out_type=jax.ShapeDtypeStruct((B, L, EMBED), jnp.float32),
    scratch_types=[
        pltpu.VMEM((ROWS_W, L), jnp.int32),
        pltpu.VMEM((NBUF, CB, L, EMBED), jnp.float32),
        pltpu.SemaphoreType.DMA,
        pltpu.SemaphoreType.DMA,
        pltpu.SemaphoreType.DMA,
        pltpu.SemaphoreType.DMA,
    ],
    compiler_params=pltpu.CompilerParams(use_tc_tiling_on_sc=False),
)
def _gather_kernel(idx_hbm, table_hbm, out_hbm, idx_all, rows, gs0, gs1, ws0, ws1):
    gsem = (gs0, gs1)
    wsem = (ws0, ws1)
    wid = lax.axis_index("s") * NC + lax.axis_index("c")
    wrow = wid * ROWS_W
    pltpu.sync_copy(idx_hbm.at[pl.ds(pl.multiple_of(wrow, ROWS_W), ROWS_W)], idx_all)

    def fire_gather(i, b):
        for j in range(CB):
            pltpu.async_copy(
                table_hbm.at[idx_all.at[i * CB + j]], rows.at[b, j], gsem[b]
            )

    def wait_gather(i, b):
        for j in range(CB):
            pltpu.make_async_copy(
                table_hbm.at[idx_all.at[i * CB + j]], rows.at[b, j], gsem[b]
            ).wait()

    def out_slice(i):
        return out_hbm.at[pl.ds(pl.multiple_of(wrow + i * CB, CB), CB)]

    def fire_write(i, b):
        pltpu.async_copy(rows.at[b], out_slice(i), wsem[b])

    def wait_write(i, b):
        pltpu.make_async_copy(rows.at[b], out_slice(i), wsem[b]).wait()

    for b in range(NBUF):
        fire_gather(b, b)

    def group(g, carry):
        for b in range(NBUF):
            i = g * NBUF + b
            wait_gather(i, b)
            fire_write(i, b)
            wait_write(i, b)
            fire_gather(i + NBUF, b)
        return carry

    lax.fori_loop(0, NG - 1, group, 0)

    for b in range(NBUF):
        i = (NG - 1) * NBUF + b
        wait_gather(i, b)
        fire_write(i, b)
    for b in range(NBUF):
        wait_write((NG - 1) * NBUF + b, b)


def kernel(indices, table):
    return _gather_kernel(indices.astype(jnp.int32), table)
